# baseline (speedup 1.0000x reference)
import functools

import jax
import jax.numpy as jnp
from jax import lax
from jax.experimental import pallas as pl
from jax.experimental.pallas import tpu as pltpu
from jax.experimental.pallas import tpu_sc as plsc

EMBED_DIM = 64
BATCH = 16384
HIST = 50
VOCAB = 1000000
TOTAL = BATCH * HIST

NUM_CORES = 2
NUM_SUBCORES = 16
NW = NUM_CORES * NUM_SUBCORES
PER_WORKER = TOTAL // NW

CHUNK = 512
N_CHUNKS = PER_WORKER // CHUNK
NBUF = 2


T1_W = 1024
T1_GRID = -(-VOCAB // T1_W)


def _t1_body(x_ref, o_ref):
    x = x_ref[...]
    x3 = x.reshape(EMBED_DIM, T1_W // 2, 2)
    even = x3[:, :, 0]
    odd = x3[:, :, 1]
    o_ref[...] = jnp.concatenate([even.T, odd.T], axis=1)


_t1 = pl.pallas_call(
    _t1_body,
    grid=(T1_GRID,),
    in_specs=[pl.BlockSpec((EMBED_DIM, T1_W), lambda i: (0, i))],
    out_specs=pl.BlockSpec((T1_W // 2, 128), lambda i: (i, 0)),
    out_shape=jax.ShapeDtypeStruct((VOCAB // 2, 128), jnp.float32),
)


def _build_gather():
    mesh = plsc.VectorSubcoreMesh(core_axis_name="c", subcore_axis_name="s")

    @functools.partial(
        pl.kernel,
        mesh=mesh,
        out_type=jax.ShapeDtypeStruct((TOTAL, EMBED_DIM), jnp.float32),
        scratch_types=[
            pltpu.VMEM((N_CHUNKS, CHUNK), jnp.int32),
            pltpu.VMEM((CHUNK, EMBED_DIM), jnp.float32),
            pltpu.VMEM((CHUNK, EMBED_DIM), jnp.float32),
            pltpu.SemaphoreType.DMA,
            pltpu.SemaphoreType.DMA,
            pltpu.SemaphoreType.DMA,
            pltpu.SemaphoreType.DMA,
        ],
        compiler_params=pltpu.CompilerParams(use_tc_tiling_on_sc=False),
    )
    def gather_kernel(idx_hbm, table_hbm, out_hbm,
                      idx_all, rows0, rows1, sg0, sg1, so0, so1):
        wid = lax.axis_index("s") * NUM_CORES + lax.axis_index("c")
        base = wid * PER_WORKER

        rows = (rows0, rows1)
        sg = (sg0, sg1)
        so = (so0, so1)

        pltpu.sync_copy(idx_hbm.at[wid], idx_all)

        pltpu.async_copy(table_hbm.at[idx_all.at[0]], rows0, sg0)
        pltpu.async_copy(table_hbm.at[idx_all.at[1]], rows1, sg1)

        def outer(i, carry):
            for b in range(NBUF):
                g = NBUF * i + b
                pltpu.make_async_copy(table_hbm.at[idx_all.at[g]],
                                      rows[b], sg[b]).wait()
                out_dma = pltpu.async_copy(
                    rows[b], out_hbm.at[pl.ds(base + g * CHUNK, CHUNK)], so[b])
                out_dma.wait()

                @pl.when(g + NBUF < N_CHUNKS)
                def _():
                    pltpu.async_copy(table_hbm.at[idx_all.at[g + NBUF]],
                                     rows[b], sg[b])
            return carry

        lax.fori_loop(0, N_CHUNKS // NBUF, outer, 0)

    return gather_kernel


_gather = _build_gather()


T2_PAIRS = 512


def _t2_body(x_ref, o_ref):
    x = x_ref[0]
    even = x[:, 0:EMBED_DIM].T
    odd = x[:, EMBED_DIM:128].T
    y = jnp.stack([even, odd], axis=2)
    o_ref[0] = y.reshape(EMBED_DIM, 2 * T2_PAIRS)


_PAIRS_PER_H = BATCH * EMBED_DIM // 128

_t2 = pl.pallas_call(
    _t2_body,
    grid=(HIST, _PAIRS_PER_H // T2_PAIRS),
    in_specs=[pl.BlockSpec((1, T2_PAIRS, 128), lambda h, j: (h, j, 0))],
    out_specs=pl.BlockSpec((1, EMBED_DIM, 2 * T2_PAIRS), lambda h, j: (h, 0, j)),
    out_shape=jax.ShapeDtypeStruct((HIST, EMBED_DIM, BATCH), jnp.float32),
)


@jax.jit
def kernel(data, table):
    idx = data.T.reshape(NW, N_CHUNKS, CHUNK).astype(jnp.int32)
    table_rm = _t1(jnp.swapaxes(table, 0, 1))
    flat = _gather(idx, table_rm.reshape(VOCAB, EMBED_DIM))
    o2 = _t2(flat.reshape(HIST, _PAIRS_PER_H, 128))
    return jnp.transpose(o2, (2, 0, 1))


# speedup vs baseline: 20.7911x; 20.7911x over previous
import functools

import jax
import jax.numpy as jnp
from jax import lax
from jax.experimental import pallas as pl
from jax.experimental.pallas import tpu as pltpu
from jax.experimental.pallas import tpu_sc as plsc

EMBED_DIM = 64
BATCH = 16384
HIST = 50
VOCAB = 1000000
TOTAL = BATCH * HIST

NUM_CORES = 2
NUM_SUBCORES = 16
NW = NUM_CORES * NUM_SUBCORES
COLS = BATCH // NW
NBUF = 2


def _build_gather():
    mesh = plsc.VectorSubcoreMesh(core_axis_name="c", subcore_axis_name="s")

    @functools.partial(
        pl.kernel,
        mesh=mesh,
        out_type=jax.ShapeDtypeStruct((TOTAL, EMBED_DIM), jnp.float32),
        scratch_types=[
            pltpu.VMEM((HIST, COLS), jnp.int32),
            pltpu.VMEM((COLS, EMBED_DIM), jnp.float32),
            pltpu.VMEM((COLS, EMBED_DIM), jnp.float32),
            pltpu.SemaphoreType.DMA,
            pltpu.SemaphoreType.DMA,
            pltpu.SemaphoreType.DMA,
            pltpu.SemaphoreType.DMA,
        ],
        compiler_params=pltpu.CompilerParams(use_tc_tiling_on_sc=False),
    )
    def gather_kernel(idx_hbm, table_hbm, out_hbm,
                      idx_all, rows0, rows1, sg0, sg1, so0, so1):
        wid = lax.axis_index("s") * NUM_CORES + lax.axis_index("c")
        col0 = wid * COLS

        rows = (rows0, rows1)
        sg = (sg0, sg1)
        so = (so0, so1)

        pltpu.sync_copy(idx_hbm.at[:, pl.ds(col0, COLS)], idx_all)

        pltpu.async_copy(table_hbm.at[idx_all.at[0]], rows0, sg0)
        pltpu.async_copy(table_hbm.at[idx_all.at[1]], rows1, sg1)

        def outer(i, carry):
            for b in range(NBUF):
                h = NBUF * i + b
                pltpu.make_async_copy(table_hbm.at[idx_all.at[h]],
                                      rows[b], sg[b]).wait()
                out_dma = pltpu.async_copy(
                    rows[b],
                    out_hbm.at[pl.ds(h * BATCH + col0, COLS)], so[b])
                out_dma.wait()

                @pl.when(h + NBUF < HIST)
                def _():
                    pltpu.async_copy(table_hbm.at[idx_all.at[h + NBUF]],
                                     rows[b], sg[b])
            return carry

        lax.fori_loop(0, HIST // NBUF, outer, 0)

    return gather_kernel


_gather = _build_gather()


@jax.jit
def kernel(data, table):
    idx_hm = data.T.astype(jnp.int32)
    flat = _gather(idx_hm, table)
    return flat.reshape(HIST, BATCH, EMBED_DIM).transpose(1, 0, 2)
